# baseline (speedup 1.0000x reference)
import functools

import jax
import jax.numpy as jnp
from jax import lax
from jax.experimental import pallas as pl
from jax.experimental.pallas import tpu as pltpu
from jax.experimental.pallas import tpu_sc as plsc

NTOK = 1000000
NHID = 64
BATCH = 4096
SEQ = 200
B = BATCH * SEQ

NUM_CORES = 2
NUM_SUBCORES = 16
NW = NUM_CORES * NUM_SUBCORES
PER_W = B // NW

GATHER = 128
N_G = 8
CHUNK = GATHER * N_G
N_CHUNKS = PER_W // CHUNK

_mesh = plsc.VectorSubcoreMesh(core_axis_name="c", subcore_axis_name="s")


@functools.partial(
    pl.kernel,
    mesh=_mesh,
    out_type=jax.ShapeDtypeStruct((B, NHID), jnp.float32),
    scratch_types=[
        pltpu.VMEM((N_G, GATHER), jnp.int32),
        pltpu.VMEM((CHUNK, NHID), jnp.float32),
        pltpu.SemaphoreType.DMA,
    ],
    compiler_params=pltpu.CompilerParams(use_tc_tiling_on_sc=False),
)
def _embed(seq_hbm, table_hbm, out_hbm, idx_v, rows_v, sem):
    wid = lax.axis_index("s") * NUM_CORES + lax.axis_index("c")
    base = wid * PER_W

    @pl.loop(0, N_CHUNKS)
    def chunk_body(c):
        off = base + c * CHUNK
        row0 = pl.multiple_of(off // GATHER, N_G)
        pltpu.sync_copy(seq_hbm.at[pl.ds(row0, N_G)], idx_v)
        cps = []
        for g in range(N_G):
            cp = pltpu.async_copy(
                table_hbm.at[idx_v.at[g]],
                rows_v.at[pl.ds(g * GATHER, GATHER)],
                sem,
            )
            cps.append(cp)
        for cp in cps:
            cp.wait()
        pltpu.sync_copy(rows_v, out_hbm.at[pl.ds(off, CHUNK)])


def kernel(seq, table):
    seq2d = seq.reshape(B // GATHER, GATHER)
    out = _embed(seq2d, table)
    return out.reshape(BATCH, SEQ, NHID)


# speedup vs baseline: 1.0156x; 1.0156x over previous
import functools

import jax
import jax.numpy as jnp
from jax import lax
from jax.experimental import pallas as pl
from jax.experimental.pallas import tpu as pltpu
from jax.experimental.pallas import tpu_sc as plsc

NTOK = 1000000
NHID = 64
BATCH = 4096
SEQ = 200
B = BATCH * SEQ

NUM_CORES = 2
NUM_SUBCORES = 16
NW = NUM_CORES * NUM_SUBCORES
PER_W = B // NW

GATHER = 128
N_G = 5
CHUNK = GATHER * N_G
N_CHUNKS = PER_W // CHUNK
N_IDX_ROWS = PER_W // GATHER
NBUF = 2

_mesh = plsc.VectorSubcoreMesh(core_axis_name="c", subcore_axis_name="s")


@functools.partial(
    pl.kernel,
    mesh=_mesh,
    out_type=jax.ShapeDtypeStruct((B, NHID), jnp.float32),
    scratch_types=[
        pltpu.VMEM((N_IDX_ROWS, GATHER), jnp.int32),
        pltpu.VMEM((NBUF, CHUNK, NHID), jnp.float32),
        pltpu.SemaphoreType.DMA,
        pltpu.SemaphoreType.DMA,
        pltpu.SemaphoreType.DMA,
        pltpu.SemaphoreType.DMA,
    ],
    compiler_params=pltpu.CompilerParams(use_tc_tiling_on_sc=False),
)
def _embed(seq_hbm, table_hbm, out_hbm, idx_v, rows_v, g0, g1, w0, w1):
    gsem = (g0, g1)
    wsem = (w0, w1)
    wid = lax.axis_index("s") * NUM_CORES + lax.axis_index("c")
    base = wid * PER_W
    row_base = pl.multiple_of(wid * N_IDX_ROWS, 8)

    pltpu.sync_copy(seq_hbm.at[pl.ds(row_base, N_IDX_ROWS)], idx_v)

    def fire(c, b):
        for g in range(N_G):
            pltpu.async_copy(
                table_hbm.at[idx_v.at[c * N_G + g]],
                rows_v.at[b, pl.ds(g * GATHER, GATHER)],
                gsem[b],
            )

    def drain_gather(b):
        pltpu.make_async_copy(
            out_hbm.at[pl.ds(0, CHUNK)], rows_v.at[b], gsem[b]
        ).wait()

    def start_write(c, b):
        pltpu.async_copy(
            rows_v.at[b], out_hbm.at[pl.ds(base + c * CHUNK, CHUNK)], wsem[b]
        )

    def drain_write(b):
        pltpu.make_async_copy(
            rows_v.at[b], out_hbm.at[pl.ds(0, CHUNK)], wsem[b]
        ).wait()

    fire(0, 0)

    @pl.loop(0, N_CHUNKS // NBUF)
    def outer(gidx):
        for b in range(NBUF):
            c = gidx * NBUF + b
            nb = (b + 1) % NBUF
            @pl.when(c + 1 < N_CHUNKS)
            def _():
                @pl.when(c + 1 >= NBUF)
                def _():
                    drain_write(nb)

                fire(c + 1, nb)

            drain_gather(b)
            start_write(c, b)

    drain_write((N_CHUNKS - 1) % NBUF)


def kernel(seq, table):
    seq2d = seq.reshape(B // GATHER, GATHER)
    out = _embed(seq2d, table)
    return out.reshape(BATCH, SEQ, NHID)


# speedup vs baseline: 1.2393x; 1.2203x over previous
import functools

import jax
import jax.numpy as jnp
from jax import lax
from jax.experimental import pallas as pl
from jax.experimental.pallas import tpu as pltpu
from jax.experimental.pallas import tpu_sc as plsc

NTOK = 1000000
NHID = 64
NPAD = 128
BATCH = 4096
SEQ = 200
B = BATCH * SEQ

NUM_CORES = 2
NUM_SUBCORES = 16
NW = NUM_CORES * NUM_SUBCORES
PER_W = B // NW

GATHER = 128
N_G = 2
CHUNK = GATHER * N_G
N_CHUNKS = PER_W // CHUNK
N_IDX_ROWS = PER_W // GATHER
NBUF = 2

_mesh = plsc.VectorSubcoreMesh(core_axis_name="c", subcore_axis_name="s")


@functools.partial(
    pl.kernel,
    mesh=_mesh,
    out_type=jax.ShapeDtypeStruct((B, NPAD), jnp.float32),
    scratch_types=[
        pltpu.VMEM((N_IDX_ROWS, GATHER), jnp.int32),
        pltpu.VMEM((NBUF, CHUNK, NPAD), jnp.float32),
        pltpu.SemaphoreType.DMA,
        pltpu.SemaphoreType.DMA,
        pltpu.SemaphoreType.DMA,
        pltpu.SemaphoreType.DMA,
    ],
    compiler_params=pltpu.CompilerParams(use_tc_tiling_on_sc=False),
)
def _embed(seq_hbm, table_hbm, out_hbm, idx_v, rows_v, g0, g1, w0, w1):
    gsem = (g0, g1)
    wsem = (w0, w1)
    wid = lax.axis_index("s") * NUM_CORES + lax.axis_index("c")
    base = wid * PER_W
    row_base = pl.multiple_of(wid * N_IDX_ROWS, 8)

    pltpu.sync_copy(seq_hbm.at[pl.ds(row_base, N_IDX_ROWS)], idx_v)

    def fire(c, b):
        for g in range(N_G):
            pltpu.async_copy(
                table_hbm.at[idx_v.at[c * N_G + g]],
                rows_v.at[b, pl.ds(g * GATHER, GATHER)],
                gsem[b],
            )

    def drain_gather(b):
        pltpu.make_async_copy(
            out_hbm.at[pl.ds(0, CHUNK)], rows_v.at[b], gsem[b]
        ).wait()

    def start_write(c, b):
        pltpu.async_copy(
            rows_v.at[b], out_hbm.at[pl.ds(base + c * CHUNK, CHUNK)], wsem[b]
        )

    def drain_write(b):
        pltpu.make_async_copy(
            rows_v.at[b], out_hbm.at[pl.ds(0, CHUNK)], wsem[b]
        ).wait()

    fire(0, 0)

    @pl.loop(0, N_CHUNKS // NBUF)
    def outer(gidx):
        for b in range(NBUF):
            c = gidx * NBUF + b
            nb = (b + 1) % NBUF
            @pl.when(c + 1 < N_CHUNKS)
            def _():
                @pl.when(c + 1 >= NBUF)
                def _():
                    drain_write(nb)

                fire(c + 1, nb)

            drain_gather(b)
            start_write(c, b)

    drain_write((N_CHUNKS - 1) % NBUF)


def kernel(seq, table):
    seq2d = seq.reshape(B // GATHER, GATHER)
    tbl128 = jnp.pad(table, ((0, 0), (0, NPAD - NHID)))
    out = _embed(seq2d, tbl128)
    return out[:, :NHID].reshape(BATCH, SEQ, NHID)
